# baseline (speedup 1.0000x reference)
import functools
import jax
import jax.numpy as jnp
from jax import lax
from jax.experimental import pallas as pl
from jax.experimental.pallas import tpu as pltpu
from jax.experimental.pallas import tpu_sc as plsc


def _gather_index_table():
    t = jnp.arange(16, dtype=jnp.int32)
    pat = (t // 2) * 128 + (t % 2)
    r = jnp.arange(512, dtype=jnp.int32)
    w = r % 8
    j = (r // 8) % 2
    i = (r // 16) % 2
    cp = r // 32
    base = w * 4096 + 8 * cp + 4 * i + 2 * j
    s = jnp.arange(4, dtype=jnp.int32)
    full = (base[:, None, None] + s[None, :, None] * 1024
            + pat[None, None, :])
    return full.reshape(-1)


def kernel(x):
    B, Cr3, H, W, Z = x.shape
    C = Cr3 // 8
    xt = (
        x.reshape(B, 4, 128, H, W, 4, 8)
        .transpose(0, 3, 4, 5, 1, 6, 2)
        .reshape(-1)
    )
    idx_table = _gather_index_table()
    n_out = B * Cr3 * H * W * Z
    mesh = plsc.VectorSubcoreMesh(core_axis_name="c", subcore_axis_name="s")

    @functools.partial(
        pl.kernel,
        mesh=mesh,
        out_type=jax.ShapeDtypeStruct((n_out,), jnp.float32),
        compiler_params=pltpu.CompilerParams(needs_layout_passes=False),
        scratch_types=[
            pltpu.VMEM((32768,), jnp.float32),
            pltpu.VMEM((32768,), jnp.float32),
            pltpu.VMEM((32768,), jnp.int32),
            pltpu.SemaphoreType.DMA,
            pltpu.SemaphoreType.DMA,
        ],
    )
    def k(x_hbm, idx_hbm, o_hbm, ibuf, obuf, idxbuf, isem, osem):
        wid = lax.axis_index("s") * 2 + lax.axis_index("c")
        pltpu.sync_copy(idx_hbm.at[...], idxbuf.at[...])

        def task_body(it, carry):
            task = it * 32 + wid
            wq = task % 4
            h = (task // 4) % 32
            cg = (task // 128) % 4
            b = task // 512
            for r2 in range(32):
                src_off = ((((b * 32 + h) * 32 + (8 * wq + r2 // 4)) * 4
                            + r2 % 4) * 4 + cg) * 1024
                pltpu.async_copy(
                    x_hbm.at[pl.ds(src_off, 1024)],
                    ibuf.at[pl.ds(r2 * 1024, 1024)],
                    isem,
                )
            pltpu.make_async_copy(
                x_hbm.at[pl.ds(0, 32768)], ibuf.at[...], isem
            ).wait()

            def gloop(r, carry2):
                dst = ((r // 16) * 16 + 2 * (r % 8) + (r // 8) % 2) * 64
                for s in range(4):
                    vec = plsc.load_gather(
                        ibuf, [idxbuf[pl.ds((r * 4 + s) * 16, 16)]])
                    obuf[pl.ds(dst + 16 * s, 16)] = vec
                return carry2

            lax.fori_loop(0, 512, gloop, 0)

            for r3 in range(32):
                off = (((b * 64 + 16 * cg + r3 // 2) * 64 + 2 * h + r3 % 2)
                       * 64 + 16 * wq) * 64
                pltpu.async_copy(
                    obuf.at[pl.ds(r3 * 1024, 1024)],
                    o_hbm.at[pl.ds(off, 1024)],
                    osem,
                )
            pltpu.make_async_copy(
                x_hbm.at[pl.ds(0, 32768)], obuf.at[...], osem
            ).wait()
            return carry

        lax.fori_loop(0, 32, task_body, 0)

    out = k(xt, idx_table)
    return out.reshape(B, C, 2 * H, 2 * W, 2 * Z)


# speedup vs baseline: 3.1533x; 3.1533x over previous
import jax
import jax.numpy as jnp
from jax import lax
from jax.experimental import pallas as pl
from jax.experimental.pallas import tpu as pltpu

_CB = 4


def _deint_kernel(x_ref, o_ref):
    v = x_ref[0]
    cb = v.shape[0]
    ss = lax.broadcasted_iota(jnp.int32, (64, 64), 0)
    ll = lax.broadcasted_iota(jnp.int32, (64, 64), 1)
    g2 = (ll == 2 * (ss % 32) + ss // 32).astype(v.dtype)
    for i in range(2):
        for j in range(2):
            a = jnp.concatenate(
                [v[:, 4 * i + 2 * j].reshape(cb * 1024, 32),
                 v[:, 4 * i + 2 * j + 1].reshape(cb * 1024, 32)],
                axis=1,
            )
            g = jnp.dot(a, g2, preferred_element_type=jnp.float32)
            o_ref[0, :, :, i : i + 1, pl.Slice(j, 32, 2), :] = (
                g.reshape(cb, 32, 1, 32, 64))


def kernel(x):
    B, Cr3, H, W, Z = x.shape
    C = Cr3 // 8
    xr = x.reshape(B, C, 8, H, W, Z)
    out = pl.pallas_call(
        _deint_kernel,
        grid=(B, C // _CB),
        in_specs=[
            pl.BlockSpec(
                (1, _CB, 8, H, W, Z),
                lambda b, c: (b, c, 0, 0, 0, 0),
            )
        ],
        out_specs=pl.BlockSpec(
            (1, _CB, H, 2, 2 * W, 2 * Z),
            lambda b, c: (b, c, 0, 0, 0, 0),
        ),
        out_shape=jax.ShapeDtypeStruct((B, C, H, 2, 2 * W, 2 * Z), x.dtype),
        compiler_params=pltpu.CompilerParams(
            dimension_semantics=("parallel", "parallel"),
        ),
    )(xr)
    return out.reshape(B, C, 2 * H, 2 * W, 2 * Z)
